# Initial kernel scaffold; baseline (speedup 1.0000x reference)
import jax
import jax.numpy as jnp
from jax.experimental import pallas as pl


def kernel(ufea, vfea, UV_adj, VU_adj, W1, b1, W2, b2, W3, b3, W4, b4, Wu, bu, Wi, bi):
    raise NotImplementedError("write your pallas kernel here")


# speedup vs baseline: 1.0783x; 1.0783x over previous
import jax
import jax.numpy as jnp
from jax.experimental import pallas as pl
from jax.experimental.pallas import tpu as pltpu

ALPHA = 0.1


def _leaky(x):
    return jnp.where(x >= 0, x, ALPHA * x)


def _dot(a, b):
    return jnp.dot(a, b, preferred_element_type=jnp.float32)


def _support_body(ufea_ref, vfea_ref, w1_ref, w2_ref, s1_ref, s2_ref):
    s1_ref[...] = _dot(ufea_ref[...], w1_ref[...])
    s2_ref[...] = _dot(vfea_ref[...], w2_ref[...])


def _gc2_body(uv_ref, s2_ref, b2_ref, w4_ref, s4_ref):
    t = _leaky(_dot(uv_ref[...], s2_ref[...]) + b2_ref[...])
    s4_ref[...] = _dot(t, w4_ref[...])


def _gc14_body(vu_ref, s1_ref, s4_ref, vfea_ref, b1_ref, b4_ref, w3_ref,
               wia_ref, wib_ref, bi_ref, s3_ref, item_ref):
    vu = vu_ref[...]
    u = _leaky(_dot(vu, s1_ref[...]) + b1_ref[...])
    s3_ref[...] = _dot(u, w3_ref[...])
    i4 = _leaky(_dot(vu, s4_ref[...]) + b4_ref[...])
    item_ref[...] = jnp.maximum(
        _dot(i4, wia_ref[...]) + _dot(vfea_ref[...], wib_ref[...]) + bi_ref[...],
        0.0)


def _gc3_body(uv_ref, s3_ref, ufea_ref, b3_ref, wua_ref, wub_ref, bu_ref,
              user_ref):
    u3 = _leaky(_dot(uv_ref[...], s3_ref[...]) + b3_ref[...])
    user_ref[...] = jnp.maximum(
        _dot(u3, wua_ref[...]) + _dot(ufea_ref[...], wub_ref[...]) + bu_ref[...],
        0.0)


def kernel(ufea, vfea, UV_adj, VU_adj, W1, b1, W2, b2, W3, b3, W4, b4, Wu, bu, Wi, bi):
    N, F = ufea.shape
    H = W1.shape[1]
    M = 512
    grid = (N // M,)

    f32 = jnp.float32
    b1r = b1.reshape(1, H)
    b2r = b2.reshape(1, H)
    b3r = b3.reshape(1, F)
    b4r = b4.reshape(1, F)
    bur = bu.reshape(1, F)
    bir = bi.reshape(1, F)
    WuA = Wu[:, :F].T
    WuB = Wu[:, F:].T
    WiA = Wi[:, :F].T
    WiB = Wi[:, F:].T

    strip_adj = pl.BlockSpec((M, N), lambda i: (i, 0))
    strip_fea = pl.BlockSpec((M, F), lambda i: (i, 0))
    full_sup = pl.BlockSpec((N, H), lambda i: (0, 0))
    small_w = pl.BlockSpec((F, F), lambda i: (0, 0))
    small_b = pl.BlockSpec((1, F), lambda i: (0, 0))
    params = pltpu.CompilerParams(dimension_semantics=("parallel",))

    S1, S2 = pl.pallas_call(
        _support_body,
        grid=grid,
        in_specs=[strip_fea, strip_fea, small_w, small_w],
        out_specs=[strip_fea, strip_fea],
        out_shape=[jax.ShapeDtypeStruct((N, H), f32)] * 2,
        compiler_params=params,
    )(ufea, vfea, W1, W2)

    S4 = pl.pallas_call(
        _gc2_body,
        grid=grid,
        in_specs=[strip_adj, full_sup, small_b, small_w],
        out_specs=strip_fea,
        out_shape=jax.ShapeDtypeStruct((N, F), f32),
        compiler_params=params,
    )(UV_adj, S2, b2r, W4)

    S3, Item = pl.pallas_call(
        _gc14_body,
        grid=grid,
        in_specs=[strip_adj, full_sup, full_sup, strip_fea, small_b, small_b,
                  small_w, small_w, small_w, small_b],
        out_specs=[strip_fea, strip_fea],
        out_shape=[jax.ShapeDtypeStruct((N, H), f32),
                   jax.ShapeDtypeStruct((N, F), f32)],
        compiler_params=params,
    )(VU_adj, S1, S4, vfea, b1r, b4r, W3, WiA, WiB, bir)

    User = pl.pallas_call(
        _gc3_body,
        grid=grid,
        in_specs=[strip_adj, full_sup, strip_fea, small_b, small_w, small_w,
                  small_b],
        out_specs=strip_fea,
        out_shape=jax.ShapeDtypeStruct((N, F), f32),
        compiler_params=params,
    )(UV_adj, S3, ufea, b3r, WuA, WuB, bur)

    return (User, Item)


# speedup vs baseline: 1.0996x; 1.0197x over previous
import jax
import jax.numpy as jnp
from jax.experimental import pallas as pl
from jax.experimental.pallas import tpu as pltpu

ALPHA = 0.1
BF16 = jnp.bfloat16


def _leaky(x):
    return jnp.where(x >= 0, x, ALPHA * x)


def _dot(a, b):
    return jnp.dot(a, b, preferred_element_type=jnp.float32)


def _support_body(ufea_ref, vfea_ref, w1_ref, w2_ref, s1_ref, s2_ref):
    s1_ref[...] = _dot(ufea_ref[...], w1_ref[...]).astype(BF16)
    s2_ref[...] = _dot(vfea_ref[...], w2_ref[...]).astype(BF16)


def _gc2_body(uv_ref, s2_ref, b2_ref, w4_ref, s4_ref):
    uv = uv_ref[...].astype(BF16)
    t = _leaky(_dot(uv, s2_ref[...]) + b2_ref[...])
    s4_ref[...] = _dot(t, w4_ref[...]).astype(BF16)


def _gc14_body(vu_ref, s1_ref, s4_ref, vfea_ref, b1_ref, b4_ref, w3_ref,
               wia_ref, wib_ref, bi_ref, s3_ref, item_ref):
    vu = vu_ref[...].astype(BF16)
    u = _leaky(_dot(vu, s1_ref[...]) + b1_ref[...])
    s3_ref[...] = _dot(u, w3_ref[...]).astype(BF16)
    i4 = _leaky(_dot(vu, s4_ref[...]) + b4_ref[...])
    item_ref[...] = jnp.maximum(
        _dot(i4, wia_ref[...]) + _dot(vfea_ref[...], wib_ref[...]) + bi_ref[...],
        0.0)


def _gc3_body(uv_ref, s3_ref, ufea_ref, b3_ref, wua_ref, wub_ref, bu_ref,
              user_ref):
    uv = uv_ref[...].astype(BF16)
    u3 = _leaky(_dot(uv, s3_ref[...]) + b3_ref[...])
    user_ref[...] = jnp.maximum(
        _dot(u3, wua_ref[...]) + _dot(ufea_ref[...], wub_ref[...]) + bu_ref[...],
        0.0)


def kernel(ufea, vfea, UV_adj, VU_adj, W1, b1, W2, b2, W3, b3, W4, b4, Wu, bu, Wi, bi):
    N, F = ufea.shape
    H = W1.shape[1]
    M = 512
    grid = (N // M,)

    f32 = jnp.float32
    b1r = b1.reshape(1, H)
    b2r = b2.reshape(1, H)
    b3r = b3.reshape(1, F)
    b4r = b4.reshape(1, F)
    bur = bu.reshape(1, F)
    bir = bi.reshape(1, F)
    WuA = Wu[:, :F].T
    WuB = Wu[:, F:].T
    WiA = Wi[:, :F].T
    WiB = Wi[:, F:].T

    strip_adj = pl.BlockSpec((M, N), lambda i: (i, 0))
    strip_fea = pl.BlockSpec((M, F), lambda i: (i, 0))
    full_sup = pl.BlockSpec((N, H), lambda i: (0, 0))
    small_w = pl.BlockSpec((F, F), lambda i: (0, 0))
    small_b = pl.BlockSpec((1, F), lambda i: (0, 0))
    params = pltpu.CompilerParams(dimension_semantics=("parallel",))

    S1, S2 = pl.pallas_call(
        _support_body,
        grid=grid,
        in_specs=[strip_fea, strip_fea, small_w, small_w],
        out_specs=[strip_fea, strip_fea],
        out_shape=[jax.ShapeDtypeStruct((N, H), BF16)] * 2,
        compiler_params=params,
    )(ufea, vfea, W1, W2)

    S4 = pl.pallas_call(
        _gc2_body,
        grid=grid,
        in_specs=[strip_adj, full_sup, small_b, small_w],
        out_specs=strip_fea,
        out_shape=jax.ShapeDtypeStruct((N, F), BF16),
        compiler_params=params,
    )(UV_adj, S2, b2r, W4)

    S3, Item = pl.pallas_call(
        _gc14_body,
        grid=grid,
        in_specs=[strip_adj, full_sup, full_sup, strip_fea, small_b, small_b,
                  small_w, small_w, small_w, small_b],
        out_specs=[strip_fea, strip_fea],
        out_shape=[jax.ShapeDtypeStruct((N, H), BF16),
                   jax.ShapeDtypeStruct((N, F), f32)],
        compiler_params=params,
    )(VU_adj, S1, S4, vfea, b1r, b4r, W3, WiA, WiB, bir)

    User = pl.pallas_call(
        _gc3_body,
        grid=grid,
        in_specs=[strip_adj, full_sup, strip_fea, small_b, small_w, small_w,
                  small_b],
        out_specs=strip_fea,
        out_shape=jax.ShapeDtypeStruct((N, F), f32),
        compiler_params=params,
    )(UV_adj, S3, ufea, b3r, WuA, WuB, bur)

    return (User, Item)


# speedup vs baseline: 1.2330x; 1.1213x over previous
import jax
import jax.numpy as jnp
from jax.experimental import pallas as pl
from jax.experimental.pallas import tpu as pltpu

ALPHA = 0.1
BF16 = jnp.bfloat16


def _leaky(x):
    return jnp.where(x >= 0, x, ALPHA * x)


def _dot(a, b):
    return jnp.dot(a, b, preferred_element_type=jnp.float32)


def _body(M, uv_ref, vu_ref, ufea_ref, vfea_ref, w1_ref, w2_ref, w3_ref,
          w4_ref, b1_ref, b2_ref, b3_ref, b4_ref, wua_ref, wub_ref, bu_ref,
          wia_ref, wib_ref, bi_ref, user_ref, item_ref,
          s1_ref, s2_ref, s3_ref, s4_ref):
    i = pl.program_id(0)

    @pl.when(i == 0)
    def _phase_a():
        s1_ref[...] = _dot(ufea_ref[...], w1_ref[...]).astype(BF16)
        s2_ref[...] = _dot(vfea_ref[...], w2_ref[...]).astype(BF16)

    @pl.when((i >= 1) & (i <= 8))
    def _phase_b():
        uv = uv_ref[...].astype(BF16)
        t = _leaky(_dot(uv, s2_ref[...]) + b2_ref[...])
        s4_ref[pl.ds((i - 1) * M, M), :] = _dot(t, w4_ref[...]).astype(BF16)

    @pl.when((i >= 9) & (i <= 16))
    def _phase_c():
        vu = vu_ref[...].astype(BF16)
        u = _leaky(_dot(vu, s1_ref[...]) + b1_ref[...])
        s3_ref[pl.ds((i - 9) * M, M), :] = _dot(u, w3_ref[...]).astype(BF16)
        i4 = _leaky(_dot(vu, s4_ref[...]) + b4_ref[...])
        vf = vfea_ref[pl.ds((i - 9) * M, M), :]
        item_ref[...] = jnp.maximum(
            _dot(i4, wia_ref[...]) + _dot(vf, wib_ref[...]) + bi_ref[...],
            0.0)

    @pl.when(i >= 17)
    def _phase_d():
        uv = uv_ref[...].astype(BF16)
        u3 = _leaky(_dot(uv, s3_ref[...]) + b3_ref[...])
        uf = ufea_ref[pl.ds((i - 17) * M, M), :]
        user_ref[...] = jnp.maximum(
            _dot(u3, wua_ref[...]) + _dot(uf, wub_ref[...]) + bu_ref[...],
            0.0)


def kernel(ufea, vfea, UV_adj, VU_adj, W1, b1, W2, b2, W3, b3, W4, b4, Wu, bu, Wi, bi):
    N, F = ufea.shape
    H = W1.shape[1]
    M = 512
    S = N // M
    grid = (1 + 3 * S,)

    f32 = jnp.float32
    b1r = b1.reshape(1, H)
    b2r = b2.reshape(1, H)
    b3r = b3.reshape(1, F)
    b4r = b4.reshape(1, F)
    bur = bu.reshape(1, F)
    bir = bi.reshape(1, F)
    WuA = Wu[:, :F].T
    WuB = Wu[:, F:].T
    WiA = Wi[:, :F].T
    WiB = Wi[:, F:].T

    def uv_idx(i):
        return (jnp.where(i >= 2 * S + 1, i - (2 * S + 1),
                          jnp.clip(i - 1, 0, S - 1)), 0)

    def vu_idx(i):
        return (jnp.clip(i - (S + 1), 0, S - 1), 0)

    def item_idx(i):
        return (jnp.clip(i - (S + 1), 0, S - 1), 0)

    def user_idx(i):
        return (jnp.clip(i - (2 * S + 1), 0, S - 1), 0)

    const2 = lambda i: (0, 0)
    strip_uv = pl.BlockSpec((M, N), uv_idx)
    strip_vu = pl.BlockSpec((M, N), vu_idx)
    full_fea = pl.BlockSpec((N, F), const2)
    small_w = pl.BlockSpec((F, F), const2)
    small_b = pl.BlockSpec((1, F), const2)

    import functools
    body = functools.partial(_body, M)

    User, Item = pl.pallas_call(
        body,
        grid=grid,
        in_specs=[strip_uv, strip_vu, full_fea, full_fea,
                  small_w, small_w, small_w, small_w,
                  small_b, small_b, small_b, small_b,
                  small_w, small_w, small_b,
                  small_w, small_w, small_b],
        out_specs=[pl.BlockSpec((M, F), user_idx),
                   pl.BlockSpec((M, F), item_idx)],
        out_shape=[jax.ShapeDtypeStruct((N, F), f32),
                   jax.ShapeDtypeStruct((N, F), f32)],
        scratch_shapes=[pltpu.VMEM((N, H), BF16), pltpu.VMEM((N, H), BF16),
                        pltpu.VMEM((N, H), BF16), pltpu.VMEM((N, H), BF16)],
        compiler_params=pltpu.CompilerParams(
            dimension_semantics=("arbitrary",)),
    )(UV_adj, VU_adj, ufea, vfea,
      W1, W2, W3, W4, b1r, b2r, b3r, b4r,
      WuA, WuB, bur, WiA, WiB, bir)

    return (User, Item)


# speedup vs baseline: 1.3270x; 1.0763x over previous
import functools

import jax
import jax.numpy as jnp
from jax.experimental import pallas as pl
from jax.experimental.pallas import tpu as pltpu

ALPHA = 0.1
BF16 = jnp.bfloat16
QSCALE = 254.0


def _leaky(x):
    return jnp.where(x >= 0, x, ALPHA * x)


def _dot(a, b):
    return jnp.dot(a, b, preferred_element_type=jnp.float32)


def _body(M, S, uv_ref, vu_ref, ufea_ref, vfea_ref,
          w1_ref, w2_ref, w3_ref, w4_ref, b1_ref, b2_ref, b3_ref, b4_ref,
          wua_ref, wub_ref, bu_ref, wia_ref, wib_ref, bi_ref,
          user_ref, item_ref,
          uvq_ref, s1_ref, s2_ref, s3_ref, s4_ref, c3_ref):
    i = pl.program_id(0)

    @pl.when(i == 0)
    def _phase_a():
        s1_ref[...] = _dot(ufea_ref[...], w1_ref[...]).astype(BF16)
        s2_ref[...] = _dot(vfea_ref[...], w2_ref[...]).astype(BF16)

    @pl.when((i >= 1) & (i <= S))
    def _phase_b():
        uv = uv_ref[...].astype(BF16)
        uvq_ref[pl.ds((i - 1) * M, M), :] = jnp.round(
            uv * QSCALE - 127.0).astype(jnp.int8)
        t = _leaky(_dot(uv, s2_ref[...]) + b2_ref[...])
        s4_ref[pl.ds((i - 1) * M, M), :] = _dot(t, w4_ref[...]).astype(BF16)

    @pl.when((i >= S + 1) & (i <= 2 * S))
    def _phase_c():
        vu = vu_ref[...].astype(BF16)
        u = _leaky(_dot(vu, s1_ref[...]) + b1_ref[...])
        s3_ref[pl.ds((i - S - 1) * M, M), :] = _dot(u, w3_ref[...]).astype(BF16)
        i4 = _leaky(_dot(vu, s4_ref[...]) + b4_ref[...])
        vf = vfea_ref[pl.ds((i - S - 1) * M, M), :]
        item_ref[...] = jnp.maximum(
            _dot(i4, wia_ref[...]) + _dot(vf, wib_ref[...]) + bi_ref[...],
            0.0)

    @pl.when(i == 2 * S + 1)
    def _colsum():
        cs = jnp.sum(s3_ref[...].astype(jnp.float32), axis=0, keepdims=True)
        c3_ref[...] = cs * (127.0 / QSCALE) + b3_ref[...]

    @pl.when(i >= 2 * S + 1)
    def _phase_d():
        q = uvq_ref[pl.ds((i - 2 * S - 1) * M, M), :].astype(BF16)
        acc = _dot(q, s3_ref[...])
        u3 = _leaky(acc * (1.0 / QSCALE) + c3_ref[...])
        uf = ufea_ref[pl.ds((i - 2 * S - 1) * M, M), :]
        user_ref[...] = jnp.maximum(
            _dot(u3, wua_ref[...]) + _dot(uf, wub_ref[...]) + bu_ref[...],
            0.0)


def kernel(ufea, vfea, UV_adj, VU_adj, W1, b1, W2, b2, W3, b3, W4, b4, Wu, bu, Wi, bi):
    N, F = ufea.shape
    H = W1.shape[1]
    M = 256
    S = N // M
    grid = (1 + 3 * S,)

    f32 = jnp.float32
    b1r = b1.reshape(1, H)
    b2r = b2.reshape(1, H)
    b3r = b3.reshape(1, F)
    b4r = b4.reshape(1, F)
    bur = bu.reshape(1, F)
    bir = bi.reshape(1, F)
    WuA = Wu[:, :F].T
    WuB = Wu[:, F:].T
    WiA = Wi[:, :F].T
    WiB = Wi[:, F:].T

    def uv_idx(i):
        return (jnp.clip(i - 1, 0, S - 1), 0)

    def vu_idx(i):
        return (jnp.clip(i - (S + 1), 0, S - 1), 0)

    def item_idx(i):
        return (jnp.clip(i - (S + 1), 0, S - 1), 0)

    def user_idx(i):
        return (jnp.clip(i - (2 * S + 1), 0, S - 1), 0)

    const2 = lambda i: (0, 0)
    full_fea = pl.BlockSpec((N, F), const2)
    small_w = pl.BlockSpec((F, F), const2)
    small_b = pl.BlockSpec((1, F), const2)

    body = functools.partial(_body, M, S)

    User, Item = pl.pallas_call(
        body,
        grid=grid,
        in_specs=[pl.BlockSpec((M, N), uv_idx),
                  pl.BlockSpec((M, N), vu_idx),
                  full_fea, full_fea,
                  small_w, small_w, small_w, small_w,
                  small_b, small_b, small_b, small_b,
                  small_w, small_w, small_b,
                  small_w, small_w, small_b],
        out_specs=[pl.BlockSpec((M, F), user_idx),
                   pl.BlockSpec((M, F), item_idx)],
        out_shape=[jax.ShapeDtypeStruct((N, F), f32),
                   jax.ShapeDtypeStruct((N, F), f32)],
        scratch_shapes=[pltpu.VMEM((N, N), jnp.int8),
                        pltpu.VMEM((N, H), BF16), pltpu.VMEM((N, H), BF16),
                        pltpu.VMEM((N, H), BF16), pltpu.VMEM((N, H), BF16),
                        pltpu.VMEM((1, F), f32)],
        compiler_params=pltpu.CompilerParams(
            dimension_semantics=("arbitrary",)),
    )(UV_adj, VU_adj, ufea, vfea,
      W1, W2, W3, W4, b1r, b2r, b3r, b4r,
      WuA, WuB, bur, WiA, WiB, bir)

    return (User, Item)


# speedup vs baseline: 1.4839x; 1.1182x over previous
import functools

import jax
import jax.numpy as jnp
from jax.experimental import pallas as pl
from jax.experimental.pallas import tpu as pltpu

ALPHA = 0.1
BF16 = jnp.bfloat16
QSCALE = 254.0


def _leaky(x):
    return jnp.where(x >= 0, x, ALPHA * x)


def _dot(a, b):
    return jnp.dot(a, b, preferred_element_type=jnp.float32)


def _body(MB, MC, SB, SC, uv_ref, vu_ref, ufea_ref, vfea_full_ref, vfea_c_ref,
          w1_ref, w2_ref, w3_ref, w4_ref, b1_ref, b2_ref, b3_ref, b4_ref,
          wua_ref, wub_ref, bu_ref, wia_ref, wib_ref, bi_ref,
          user_ref, item_ref,
          uvq_ref, s1_ref, s2_ref, s3_ref, s4_ref, c3_ref):
    i = pl.program_id(0)
    c0 = 1 + SB
    d0 = 1 + SB + SC

    @pl.when(i == 0)
    def _phase_a():
        s2_ref[...] = _dot(vfea_full_ref[...], w2_ref[...]).astype(BF16)

    @pl.when((i >= 1) & (i < c0))
    def _phase_b():
        r = (i - 1) * MB
        uv = uv_ref[...].astype(BF16)
        uvq_ref[pl.ds(r, MB), :] = jnp.round(
            uv * QSCALE - 127.0).astype(jnp.int8)
        t = _leaky(_dot(uv, s2_ref[...]) + b2_ref[...])
        s4_ref[pl.ds(r, MB), :] = _dot(t, w4_ref[...]).astype(BF16)
        s1_ref[pl.ds(r, MB), :] = _dot(ufea_ref[...], w1_ref[...]).astype(BF16)

    @pl.when((i >= c0) & (i < d0))
    def _phase_c():
        r = (i - c0) * MC
        vu = vu_ref[...].astype(BF16)
        u = _leaky(_dot(vu, s1_ref[...]) + b1_ref[...])
        s3_ref[pl.ds(r, MC), :] = _dot(u, w3_ref[...]).astype(BF16)
        i4 = _leaky(_dot(vu, s4_ref[...]) + b4_ref[...])
        item_ref[...] = jnp.maximum(
            _dot(i4, wia_ref[...]) + _dot(vfea_c_ref[...], wib_ref[...])
            + bi_ref[...], 0.0)

    @pl.when(i == d0)
    def _colsum():
        cs = jnp.sum(s3_ref[...].astype(jnp.float32), axis=0, keepdims=True)
        c3_ref[...] = cs * (127.0 / QSCALE) + b3_ref[...]

    @pl.when(i >= d0)
    def _phase_d():
        r = (i - d0) * MB
        q = uvq_ref[pl.ds(r, MB), :].astype(BF16)
        acc = _dot(q, s3_ref[...])
        u3 = _leaky(acc * (1.0 / QSCALE) + c3_ref[...])
        user_ref[...] = jnp.maximum(
            _dot(u3, wua_ref[...]) + _dot(ufea_ref[...], wub_ref[...])
            + bu_ref[...], 0.0)


def kernel(ufea, vfea, UV_adj, VU_adj, W1, b1, W2, b2, W3, b3, W4, b4, Wu, bu, Wi, bi):
    N, F = ufea.shape
    H = W1.shape[1]
    MB = 512
    MC = 256
    SB = N // MB
    SC = N // MC
    grid = (1 + 2 * SB + SC,)
    c0 = 1 + SB
    d0 = 1 + SB + SC

    f32 = jnp.float32
    b1r = b1.reshape(1, H)
    b2r = b2.reshape(1, H)
    b3r = b3.reshape(1, F)
    b4r = b4.reshape(1, F)
    bur = bu.reshape(1, F)
    bir = bi.reshape(1, F)
    WuA = Wu[:, :F].T
    WuB = Wu[:, F:].T
    WiA = Wi[:, :F].T
    WiB = Wi[:, F:].T

    def uv_idx(i):
        return (jnp.clip(i - 1, 0, SB - 1), 0)

    def vu_idx(i):
        return (jnp.clip(i - c0, 0, SC - 1), 0)

    def ufea_idx(i):
        return (jnp.where(i >= d0, i - d0, jnp.clip(i - 1, 0, SB - 1)), 0)

    def vfea_c_idx(i):
        return (jnp.clip(i - c0, 0, SC - 1), 0)

    def user_idx(i):
        return (jnp.clip(i - d0, 0, SB - 1), 0)

    def item_idx(i):
        return (jnp.clip(i - c0, 0, SC - 1), 0)

    const2 = lambda i: (0, 0)
    small_w = pl.BlockSpec((F, F), const2)
    small_b = pl.BlockSpec((1, F), const2)

    body = functools.partial(_body, MB, MC, SB, SC)

    User, Item = pl.pallas_call(
        body,
        grid=grid,
        in_specs=[pl.BlockSpec((MB, N), uv_idx),
                  pl.BlockSpec((MC, N), vu_idx),
                  pl.BlockSpec((MB, F), ufea_idx),
                  pl.BlockSpec((N, F), const2),
                  pl.BlockSpec((MC, F), vfea_c_idx),
                  small_w, small_w, small_w, small_w,
                  small_b, small_b, small_b, small_b,
                  small_w, small_w, small_b,
                  small_w, small_w, small_b],
        out_specs=[pl.BlockSpec((MB, F), user_idx),
                   pl.BlockSpec((MC, F), item_idx)],
        out_shape=[jax.ShapeDtypeStruct((N, F), f32),
                   jax.ShapeDtypeStruct((N, F), f32)],
        scratch_shapes=[pltpu.VMEM((N, N), jnp.int8),
                        pltpu.VMEM((N, H), BF16), pltpu.VMEM((N, H), BF16),
                        pltpu.VMEM((N, H), BF16), pltpu.VMEM((N, H), BF16),
                        pltpu.VMEM((1, F), f32)],
        compiler_params=pltpu.CompilerParams(
            dimension_semantics=("arbitrary",)),
    )(UV_adj, VU_adj, ufea, vfea, vfea,
      W1, W2, W3, W4, b1r, b2r, b3r, b4r,
      WuA, WuB, bur, WiA, WiB, bir)

    return (User, Item)
